# baseline (speedup 1.0000x reference)
import functools

import jax
import jax.numpy as jnp
from jax import lax
from jax.experimental import pallas as pl
from jax.experimental.pallas import tpu as pltpu
from jax.experimental.pallas import tpu_sc as plsc

_B, _P, _N, _K, _D = 1024, 20, 20, 20, 32
_W1, _W2, _W3 = 0.5, 0.1, 0.5
_NC, _NS = 2, 16
_NW = _NC * _NS
_UPW = _B // _NW
_IPW = _B * _P // _NW
_GCH = 128
_NCHK = _IPW // _GCH
_MG = 16

_sc_out_type = (
    jax.ShapeDtypeStruct((_B, _D), jnp.float32),
    jax.ShapeDtypeStruct((_B * _P, _D), jnp.float32),
    jax.ShapeDtypeStruct((_B * _N, _D), jnp.float32),
    jax.ShapeDtypeStruct((_B * _K, _D), jnp.float32),
    jax.ShapeDtypeStruct((_B * _K, _D), jnp.float32),
    jax.ShapeDtypeStruct((_B, _MG), jnp.float32),
    jax.ShapeDtypeStruct((_B * _P, _MG), jnp.float32),
    jax.ShapeDtypeStruct((_B, _MG), jnp.float32),
)

_sc_scratch = (
    pltpu.VMEM((_UPW,), jnp.int32),
    pltpu.VMEM((_IPW,), jnp.int32),
    pltpu.VMEM((_IPW,), jnp.int32),
    pltpu.VMEM((_IPW,), jnp.int32),
    pltpu.VMEM((_IPW,), jnp.int32),
    pltpu.VMEM((_UPW,), jnp.int32),
    pltpu.VMEM((_IPW,), jnp.int32),
    pltpu.VMEM((_UPW, _D), jnp.float32),
    pltpu.VMEM((_IPW, _D), jnp.float32),
    pltpu.VMEM((_IPW, _D), jnp.float32),
    pltpu.VMEM((_IPW, _D), jnp.float32),
    pltpu.VMEM((_IPW, _D), jnp.float32),
    pltpu.VMEM((_UPW, _MG), jnp.float32),
    pltpu.VMEM((_IPW, _MG), jnp.float32),
    pltpu.VMEM((_UPW, _MG), jnp.float32),
    pltpu.SemaphoreType.DMA,
)


@functools.cache
def _make_sc_gather():
    mesh = plsc.VectorSubcoreMesh(core_axis_name="c", subcore_axis_name="s")
    return functools.partial(
        pl.kernel, mesh=mesh, out_type=_sc_out_type,
        scratch_types=list(_sc_scratch),
        compiler_params=pltpu.CompilerParams(use_tc_tiling_on_sc=False),
    )(_sc_gather_body)


def _sc_gather_body(tu_h, pos_h, neg_h, nb_h, nnb_h, u2e_h, v2e_h,
                    muv_h, mvv_h, muu_h,
                    u_o, p_o, n_o, nb_o, nnb_o, mu_o, mv_o, ms_o,
                    iu, ip, ineg, inb, innb, iur, ipr,
                    ru, rp, rn, rnb, rnnb, rmu, rmv, rms, sem):
    wid = lax.axis_index("s") * _NC + lax.axis_index("c")
    bu = wid * _UPW
    bi = wid * _IPW

    pltpu.sync_copy(tu_h.at[pl.ds(bu, _UPW)], iu)
    pltpu.sync_copy(pos_h.at[pl.ds(bi, _IPW)], ip)
    pltpu.sync_copy(neg_h.at[pl.ds(bi, _IPW)], ineg)
    pltpu.sync_copy(nb_h.at[pl.ds(bi, _IPW)], inb)
    pltpu.sync_copy(nnb_h.at[pl.ds(bi, _IPW)], innb)

    for t in range(_UPW // 16):
        s = pl.ds(t * 16, 16)
        iur[s] = lax.shift_right_logical(iu[s], 4)
    for t in range(_IPW // 16):
        s = pl.ds(t * 16, 16)
        ipr[s] = lax.shift_right_logical(ip[s], 4)

    copies = [
        pltpu.async_copy(u2e_h.at[iu], ru, sem),
        pltpu.async_copy(muv_h.at[iur], rmu, sem),
        pltpu.async_copy(muu_h.at[iur], rms, sem),
    ]
    for k in range(_NCHK):
        sl = pl.ds(k * _GCH, _GCH)
        copies.append(pltpu.async_copy(v2e_h.at[ip.at[sl]], rp.at[sl], sem))
        copies.append(pltpu.async_copy(v2e_h.at[ineg.at[sl]], rn.at[sl], sem))
        copies.append(pltpu.async_copy(u2e_h.at[inb.at[sl]], rnb.at[sl], sem))
        copies.append(pltpu.async_copy(u2e_h.at[innb.at[sl]], rnnb.at[sl], sem))
        copies.append(pltpu.async_copy(mvv_h.at[ipr.at[sl]], rmv.at[sl], sem))
    for c in copies:
        c.wait()

    pltpu.sync_copy(ru, u_o.at[pl.ds(bu, _UPW)])
    pltpu.sync_copy(rp, p_o.at[pl.ds(bi, _IPW)])
    pltpu.sync_copy(rn, n_o.at[pl.ds(bi, _IPW)])
    pltpu.sync_copy(rnb, nb_o.at[pl.ds(bi, _IPW)])
    pltpu.sync_copy(rnnb, nnb_o.at[pl.ds(bi, _IPW)])
    pltpu.sync_copy(rmu, mu_o.at[pl.ds(bu, _UPW)])
    pltpu.sync_copy(rmv, mv_o.at[pl.ds(bi, _IPW)])
    pltpu.sync_copy(rms, ms_o.at[pl.ds(bu, _UPW)])


_CHUNK = 128
_NSTEP = _B // _CHUNK


def _lane_select(rows, idx):
    col = jnp.bitwise_and(idx, _MG - 1)
    onehot = (col[..., None] ==
              lax.broadcasted_iota(jnp.int32, col.shape + (_MG,),
                                   len(col.shape)))
    return jnp.sum(jnp.where(onehot, rows, 0.0), axis=-1)


def _loss_body(u_ref, pos_ref, neg_ref, nb_ref, nnb_ref,
               mu_ref, mv_ref, ms_ref, tu_ref, pi_ref, out_ref):
    i = pl.program_id(0)
    u = u_ref[...]
    pos = pos_ref[...]
    neg = neg_ref[...]
    nb = nb_ref[...]
    nnb = nnb_ref[...]
    tu = tu_ref[...]
    pi = pi_ref[...]
    mu = _lane_select(mu_ref[...][:, None, :], tu)
    ms = _lane_select(ms_ref[...][:, None, :], tu)
    mv = _lane_select(mv_ref[...], pi)

    ud = u[:, None, :]
    d1 = jnp.sum((pos - ud) ** 2, axis=-1)
    d2 = jnp.sum((neg - ud) ** 2, axis=-1)
    d1n = jnp.sum((nb - ud) ** 2, axis=-1)
    d2n = jnp.sum((nnb - ud) ** 2, axis=-1)

    uv = jnp.sum(jnp.maximum((mu + d1)[:, :, None] - d2[:, None, :], 0.0))
    uu = jnp.sum(jnp.maximum((ms + d1n)[:, :, None] - d2n[:, None, :], 0.0))

    pn = lax.dot_general(pos, neg, (((2,), (2,)), ((0,), (0,))),
                         preferred_element_type=jnp.float32)
    p2 = jnp.sum(pos * pos, axis=-1)
    n2 = jnp.sum(neg * neg, axis=-1)
    z = (mv - p2)[:, :, None] - (n2[:, None, :] - 2.0 * pn)
    vv = jnp.sum(jnp.maximum(d1[:, :, None] + z.reshape(_CHUNK, 1, _P * _N), 0.0))

    am = jnp.sum(mu) / _B + jnp.sum(mv) / (_B * _P) + jnp.sum(ms) / _B
    part = uv + _W1 * vv + _W3 * uu + _W2 * am

    @pl.when(i == 0)
    def _init():
        out_ref[0, 0] = 0.0

    out_ref[0, 0] += part


_loss_call = pl.pallas_call(
    _loss_body,
    grid=(_NSTEP,),
    in_specs=[
        pl.BlockSpec((_CHUNK, _D), lambda i: (i, 0)),
        pl.BlockSpec((_CHUNK, _P, _D), lambda i: (i, 0, 0)),
        pl.BlockSpec((_CHUNK, _N, _D), lambda i: (i, 0, 0)),
        pl.BlockSpec((_CHUNK, _K, _D), lambda i: (i, 0, 0)),
        pl.BlockSpec((_CHUNK, _K, _D), lambda i: (i, 0, 0)),
        pl.BlockSpec((_CHUNK, _MG), lambda i: (i, 0)),
        pl.BlockSpec((_CHUNK, _P, _MG), lambda i: (i, 0, 0)),
        pl.BlockSpec((_CHUNK, _MG), lambda i: (i, 0)),
        pl.BlockSpec((_CHUNK, 1), lambda i: (i, 0)),
        pl.BlockSpec((_CHUNK, _P), lambda i: (i, 0)),
    ],
    out_specs=pl.BlockSpec((1, 1), lambda i: (0, 0), memory_space=pltpu.SMEM),
    out_shape=jax.ShapeDtypeStruct((1, 1), jnp.float32),
)


def kernel(train_u, pos_idx, neg_idx, nb_idx, non_nb_idx, u2e, v2e,
           margin_uv, margin_vv, margin_uu):
    u_emb, pos_e, neg_e, nb_e, nnb_e, m_uv, m_vv, m_uu = _make_sc_gather()(
        train_u, pos_idx.reshape(-1), neg_idx.reshape(-1), nb_idx.reshape(-1),
        non_nb_idx.reshape(-1), u2e, v2e,
        margin_uv.reshape(-1, _MG), margin_vv.reshape(-1, _MG),
        margin_uu.reshape(-1, _MG))
    total = _loss_call(
        u_emb,
        pos_e.reshape(_B, _P, _D),
        neg_e.reshape(_B, _N, _D),
        nb_e.reshape(_B, _K, _D),
        nnb_e.reshape(_B, _K, _D),
        m_uv,
        m_vv.reshape(_B, _P, _MG),
        m_uu,
        train_u.reshape(_B, 1),
        pos_idx,
    )
    return total[0, 0]


# speedup vs baseline: 1.3986x; 1.3986x over previous
import functools

import jax
import jax.numpy as jnp
from jax import lax
from jax.experimental import pallas as pl
from jax.experimental.pallas import tpu as pltpu
from jax.experimental.pallas import tpu_sc as plsc

_B, _P, _N, _K, _D = 1024, 20, 20, 20, 32
_W1, _W2, _W3 = 0.5, 0.1, 0.5
_NC, _NS = 2, 16
_NW = _NC * _NS
_UPW = _B // _NW
_IPW = _B * _P // _NW
_GCH = 128
_NCHK = _IPW // _GCH
_MG = 16
_NV = 100000

_sc_out_type = (
    jax.ShapeDtypeStruct((_B, _D), jnp.float32),
    jax.ShapeDtypeStruct((_B * _P, _D), jnp.float32),
    jax.ShapeDtypeStruct((_B * _N, _D), jnp.float32),
    jax.ShapeDtypeStruct((_B * _K, _D), jnp.float32),
    jax.ShapeDtypeStruct((_B * _K, _D), jnp.float32),
    jax.ShapeDtypeStruct((_B, _MG), jnp.float32),
    jax.ShapeDtypeStruct((_B * _P, _MG), jnp.float32),
    jax.ShapeDtypeStruct((_B, _MG), jnp.float32),
)

_sc_scratch = (
    pltpu.VMEM((_UPW,), jnp.int32),
    pltpu.VMEM((_IPW,), jnp.int32),
    pltpu.VMEM((_IPW,), jnp.int32),
    pltpu.VMEM((_IPW,), jnp.int32),
    pltpu.VMEM((_IPW,), jnp.int32),
    pltpu.VMEM((_UPW,), jnp.int32),
    pltpu.VMEM((_IPW,), jnp.int32),
    pltpu.VMEM((_UPW, _D), jnp.float32),
    pltpu.VMEM((_IPW, _D), jnp.float32),
    pltpu.VMEM((_IPW, _D), jnp.float32),
    pltpu.VMEM((_IPW, _D), jnp.float32),
    pltpu.VMEM((_IPW, _D), jnp.float32),
    pltpu.VMEM((_UPW, _MG), jnp.float32),
    pltpu.VMEM((_IPW, _MG), jnp.float32),
    pltpu.VMEM((_UPW, _MG), jnp.float32),
    pltpu.SemaphoreType.DMA,
)


@functools.cache
def _make_sc_gather():
    mesh = plsc.VectorSubcoreMesh(core_axis_name="c", subcore_axis_name="s")
    return functools.partial(
        pl.kernel, mesh=mesh, out_type=_sc_out_type,
        scratch_types=list(_sc_scratch),
        compiler_params=pltpu.CompilerParams(use_tc_tiling_on_sc=False),
    )(_sc_gather_body)


def _sc_gather_body(tu_h, pos_h, neg_h, nb_h, nnb_h, u2e_h, v2e_h,
                    muv_h, mvv_h, muu_h,
                    u_o, p_o, n_o, nb_o, nnb_o, mu_o, mv_o, ms_o,
                    iu, ip, ineg, inb, innb, iur, ipr,
                    ru, rp, rn, rnb, rnnb, rmu, rmv, rms, sem):
    wid = lax.axis_index("s") * _NC + lax.axis_index("c")
    bu = wid * _UPW
    bi = wid * _IPW

    u2e, v2e = u2e_h, v2e_h
    muv, mvv, muu = muv_h, mvv_h, muu_h

    pltpu.sync_copy(tu_h.at[pl.ds(bu, _UPW)], iu)
    pltpu.sync_copy(pos_h.at[pl.ds(bi, _IPW)], ip)
    pltpu.sync_copy(neg_h.at[pl.ds(bi, _IPW)], ineg)
    pltpu.sync_copy(nb_h.at[pl.ds(bi, _IPW)], inb)
    pltpu.sync_copy(nnb_h.at[pl.ds(bi, _IPW)], innb)

    for t in range(_UPW // 16):
        s = pl.ds(t * 16, 16)
        iur[s] = lax.shift_right_logical(iu[s], 4)
    for t in range(_IPW // 16):
        s = pl.ds(t * 16, 16)
        ipr[s] = lax.shift_right_logical(ip[s], 4)

    copies = [
        pltpu.async_copy(u2e.at[iu], ru, sem),
        pltpu.async_copy(muv.at[iur], rmu, sem),
        pltpu.async_copy(muu.at[iur], rms, sem),
    ]
    for k in range(_NCHK):
        sl = pl.ds(k * _GCH, _GCH)
        copies.append(pltpu.async_copy(v2e.at[ip.at[sl]], rp.at[sl], sem))
        copies.append(pltpu.async_copy(v2e.at[ineg.at[sl]], rn.at[sl], sem))
        copies.append(pltpu.async_copy(u2e.at[inb.at[sl]], rnb.at[sl], sem))
        copies.append(pltpu.async_copy(u2e.at[innb.at[sl]], rnnb.at[sl], sem))
        copies.append(pltpu.async_copy(mvv.at[ipr.at[sl]], rmv.at[sl], sem))
    for c in copies:
        c.wait()

    pltpu.sync_copy(ru, u_o.at[pl.ds(bu, _UPW)])
    pltpu.sync_copy(rp, p_o.at[pl.ds(bi, _IPW)])
    pltpu.sync_copy(rn, n_o.at[pl.ds(bi, _IPW)])
    pltpu.sync_copy(rnb, nb_o.at[pl.ds(bi, _IPW)])
    pltpu.sync_copy(rnnb, nnb_o.at[pl.ds(bi, _IPW)])
    pltpu.sync_copy(rmu, mu_o.at[pl.ds(bu, _UPW)])
    pltpu.sync_copy(rmv, mv_o.at[pl.ds(bi, _IPW)])
    pltpu.sync_copy(rms, ms_o.at[pl.ds(bu, _UPW)])


_CHUNK = 256
_NSTEP = _B // _CHUNK


def _loss_body(u_ref, pos_ref, neg_ref, nb_ref, nnb_ref,
               mu_ref, mv_ref, ms_ref, tu_ref, pi_ref, out_ref, mv_s):
    i = pl.program_id(0)
    uT = u_ref[...]
    posT = pos_ref[...]
    negT = neg_ref[...]
    nbT = nb_ref[...]
    nnbT = nnb_ref[...]
    tu = tu_ref[...]
    pi = pi_ref[...]

    colu = jnp.bitwise_and(tu, _MG - 1)
    iota_u = lax.broadcasted_iota(jnp.int32, (_MG, _CHUNK), 0)
    oh_u = iota_u == colu
    mu = jnp.sum(jnp.where(oh_u, mu_ref[...], 0.0), axis=0, keepdims=True)
    ms = jnp.sum(jnp.where(oh_u, ms_ref[...], 0.0), axis=0, keepdims=True)
    colp = jnp.bitwise_and(pi, _MG - 1)
    iota_p = lax.broadcasted_iota(jnp.int32, (_P, _MG, _CHUNK), 1)
    oh_p = iota_p == colp[:, None, :]
    mv = jnp.sum(jnp.where(oh_p, mv_ref[...], 0.0), axis=1)

    ub = uT[None]
    d1 = jnp.sum((posT - ub) ** 2, axis=1)
    d2 = jnp.sum((negT - ub) ** 2, axis=1)
    d1n = jnp.sum((nbT - ub) ** 2, axis=1)
    d2n = jnp.sum((nnbT - ub) ** 2, axis=1)

    uv = jnp.sum(jnp.maximum((mu + d1)[:, None, :] - d2[None, :, :], 0.0))
    uu = jnp.sum(jnp.maximum((ms + d1n)[:, None, :] - d2n[None, :, :], 0.0))

    mv_s[...] = mv

    def jbody(j, acc):
        pj = pos_ref[pl.ds(j, 1)]
        dist2j = jnp.sum((negT - pj) ** 2, axis=1)
        mvj = mv_s[pl.ds(j, 1)]
        zj = mvj - dist2j
        hj = jnp.maximum(d1[:, None, :] + zj[None, :, :], 0.0)
        return acc + jnp.sum(hj)

    vv = lax.fori_loop(0, _P, jbody, jnp.float32(0.0))

    am = jnp.sum(mu) / _B + jnp.sum(mv) / (_B * _P) + jnp.sum(ms) / _B
    part = uv + _W1 * vv + _W3 * uu + _W2 * am

    @pl.when(i == 0)
    def _init():
        out_ref[0, 0] = 0.0

    out_ref[0, 0] += part


_loss_call = pl.pallas_call(
    _loss_body,
    grid=(_NSTEP,),
    in_specs=[
        pl.BlockSpec((_D, _CHUNK), lambda i: (0, i)),
        pl.BlockSpec((_P, _D, _CHUNK), lambda i: (0, 0, i)),
        pl.BlockSpec((_N, _D, _CHUNK), lambda i: (0, 0, i)),
        pl.BlockSpec((_K, _D, _CHUNK), lambda i: (0, 0, i)),
        pl.BlockSpec((_K, _D, _CHUNK), lambda i: (0, 0, i)),
        pl.BlockSpec((_MG, _CHUNK), lambda i: (0, i)),
        pl.BlockSpec((_P, _MG, _CHUNK), lambda i: (0, 0, i)),
        pl.BlockSpec((_MG, _CHUNK), lambda i: (0, i)),
        pl.BlockSpec((1, _CHUNK), lambda i: (0, i)),
        pl.BlockSpec((_P, _CHUNK), lambda i: (0, i)),
    ],
    out_specs=pl.BlockSpec((1, 1), lambda i: (0, 0), memory_space=pltpu.SMEM),
    out_shape=jax.ShapeDtypeStruct((1, 1), jnp.float32),
    scratch_shapes=[pltpu.VMEM((_P, _CHUNK), jnp.float32)],
)


def kernel(train_u, pos_idx, neg_idx, nb_idx, non_nb_idx, u2e, v2e,
           margin_uv, margin_vv, margin_uu):
    u_emb, pos_e, neg_e, nb_e, nnb_e, m_uv, m_vv, m_uu = _make_sc_gather()(
        train_u, pos_idx.reshape(-1), neg_idx.reshape(-1), nb_idx.reshape(-1),
        non_nb_idx.reshape(-1), u2e, v2e,
        margin_uv.reshape(-1, _MG), margin_vv.reshape(-1, _MG),
        margin_uu.reshape(-1, _MG))
    total = _loss_call(
        u_emb.T,
        pos_e.reshape(_B, _P, _D).transpose(1, 2, 0),
        neg_e.reshape(_B, _N, _D).transpose(1, 2, 0),
        nb_e.reshape(_B, _K, _D).transpose(1, 2, 0),
        nnb_e.reshape(_B, _K, _D).transpose(1, 2, 0),
        m_uv.T,
        m_vv.reshape(_B, _P, _MG).transpose(1, 2, 0),
        m_uu.T,
        train_u.reshape(1, _B),
        pos_idx.T,
    )
    return total[0, 0]


# speedup vs baseline: 1.4881x; 1.0640x over previous
import functools

import jax
import jax.numpy as jnp
from jax import lax
from jax.experimental import pallas as pl
from jax.experimental.pallas import tpu as pltpu
from jax.experimental.pallas import tpu_sc as plsc

_B, _P, _N, _K, _D = 1024, 20, 20, 20, 32
_W1, _W2, _W3 = 0.5, 0.1, 0.5
_NC, _NS = 2, 16
_NW = _NC * _NS
_UPW = _B // _NW
_IPW = _B * _P // _NW
_GCH = 128
_NCHK = _IPW // _GCH
_MG = 16
_NV = 100000

_sc_out_type = (
    jax.ShapeDtypeStruct((_B, _D), jnp.float32),
    jax.ShapeDtypeStruct((_B * _P, _D), jnp.float32),
    jax.ShapeDtypeStruct((_B * _N, _D), jnp.float32),
    jax.ShapeDtypeStruct((_B * _K, _D), jnp.float32),
    jax.ShapeDtypeStruct((_B * _K, _D), jnp.float32),
    jax.ShapeDtypeStruct((_B, _MG), jnp.float32),
    jax.ShapeDtypeStruct((_B * _P, _MG), jnp.float32),
    jax.ShapeDtypeStruct((_B, _MG), jnp.float32),
)

_sc_scratch = (
    pltpu.VMEM((_UPW,), jnp.int32),
    pltpu.VMEM((_IPW,), jnp.int32),
    pltpu.VMEM((_IPW,), jnp.int32),
    pltpu.VMEM((_IPW,), jnp.int32),
    pltpu.VMEM((_IPW,), jnp.int32),
    pltpu.VMEM((_UPW,), jnp.int32),
    pltpu.VMEM((_IPW,), jnp.int32),
    pltpu.VMEM((_UPW, _D), jnp.float32),
    pltpu.VMEM((_IPW, _D), jnp.float32),
    pltpu.VMEM((_IPW, _D), jnp.float32),
    pltpu.VMEM((_IPW, _D), jnp.float32),
    pltpu.VMEM((_IPW, _D), jnp.float32),
    pltpu.VMEM((_UPW, _MG), jnp.float32),
    pltpu.VMEM((_IPW, _MG), jnp.float32),
    pltpu.VMEM((_UPW, _MG), jnp.float32),
    pltpu.SemaphoreType.DMA,
)


@functools.cache
def _make_sc_gather():
    mesh = plsc.VectorSubcoreMesh(core_axis_name="c", subcore_axis_name="s")
    return functools.partial(
        pl.kernel, mesh=mesh, out_type=_sc_out_type,
        scratch_types=list(_sc_scratch),
        compiler_params=pltpu.CompilerParams(use_tc_tiling_on_sc=False),
    )(_sc_gather_body)


def _sc_gather_body(tu_h, pos_h, neg_h, nb_h, nnb_h, u2e_h, v2e_h,
                    muv_h, mvv_h, muu_h,
                    u_o, p_o, n_o, nb_o, nnb_o, mu_o, mv_o, ms_o,
                    iu, ip, ineg, inb, innb, iur, ipr,
                    ru, rp, rn, rnb, rnnb, rmu, rmv, rms, sem):
    wid = lax.axis_index("s") * _NC + lax.axis_index("c")
    bu = wid * _UPW
    bi = wid * _IPW

    u2e, v2e = u2e_h, v2e_h
    muv, mvv, muu = muv_h, mvv_h, muu_h

    pltpu.sync_copy(tu_h.at[pl.ds(bu, _UPW)], iu)
    pltpu.sync_copy(pos_h.at[pl.ds(bi, _IPW)], ip)
    pltpu.sync_copy(neg_h.at[pl.ds(bi, _IPW)], ineg)
    pltpu.sync_copy(nb_h.at[pl.ds(bi, _IPW)], inb)
    pltpu.sync_copy(nnb_h.at[pl.ds(bi, _IPW)], innb)

    for t in range(_UPW // 16):
        s = pl.ds(t * 16, 16)
        iur[s] = lax.shift_right_logical(iu[s], 4)
    for t in range(_IPW // 16):
        s = pl.ds(t * 16, 16)
        ipr[s] = lax.shift_right_logical(ip[s], 4)

    copies = [
        pltpu.async_copy(u2e.at[iu], ru, sem),
        pltpu.async_copy(muv.at[iur], rmu, sem),
        pltpu.async_copy(muu.at[iur], rms, sem),
    ]
    for k in range(_NCHK):
        sl = pl.ds(k * _GCH, _GCH)
        copies.append(pltpu.async_copy(v2e.at[ip.at[sl]], rp.at[sl], sem))
        copies.append(pltpu.async_copy(v2e.at[ineg.at[sl]], rn.at[sl], sem))
        copies.append(pltpu.async_copy(u2e.at[inb.at[sl]], rnb.at[sl], sem))
        copies.append(pltpu.async_copy(u2e.at[innb.at[sl]], rnnb.at[sl], sem))
        copies.append(pltpu.async_copy(mvv.at[ipr.at[sl]], rmv.at[sl], sem))
    for c in copies:
        c.wait()

    pltpu.sync_copy(ru, u_o.at[pl.ds(bu, _UPW)])
    pltpu.sync_copy(rp, p_o.at[pl.ds(bi, _IPW)])
    pltpu.sync_copy(rn, n_o.at[pl.ds(bi, _IPW)])
    pltpu.sync_copy(rnb, nb_o.at[pl.ds(bi, _IPW)])
    pltpu.sync_copy(rnnb, nnb_o.at[pl.ds(bi, _IPW)])
    pltpu.sync_copy(rmu, mu_o.at[pl.ds(bu, _UPW)])
    pltpu.sync_copy(rmv, mv_o.at[pl.ds(bi, _IPW)])
    pltpu.sync_copy(rms, ms_o.at[pl.ds(bu, _UPW)])


_CHUNK = 256
_NSTEP = _B // _CHUNK


def _loss_body(u_ref, pos_ref, neg_ref, nb_ref, nnb_ref,
               mu_ref, mv_ref, ms_ref, tu_ref, pi_ref, out_ref, mv_s):
    i = pl.program_id(0)
    uT = u_ref[...]
    posT = pos_ref[...]
    negT = neg_ref[...]
    nbT = nb_ref[...]
    nnbT = nnb_ref[...]
    tu = tu_ref[...]
    pi = pi_ref[...]

    colu = jnp.bitwise_and(tu, _MG - 1)
    iota_u = lax.broadcasted_iota(jnp.int32, (_MG, _CHUNK), 0)
    oh_u = iota_u == colu
    mu = jnp.sum(jnp.where(oh_u, mu_ref[...], 0.0), axis=0, keepdims=True)
    ms = jnp.sum(jnp.where(oh_u, ms_ref[...], 0.0), axis=0, keepdims=True)
    colp = jnp.bitwise_and(pi, _MG - 1)
    iota_p = lax.broadcasted_iota(jnp.int32, (_P, _MG, _CHUNK), 1)
    oh_p = iota_p == colp[:, None, :]
    mv = jnp.sum(jnp.where(oh_p, mv_ref[...], 0.0), axis=1)

    ub = uT[None]
    d1 = jnp.sum((posT - ub) ** 2, axis=1)
    d2 = jnp.sum((negT - ub) ** 2, axis=1)
    d1n = jnp.sum((nbT - ub) ** 2, axis=1)
    d2n = jnp.sum((nnbT - ub) ** 2, axis=1)

    uv = jnp.sum(jnp.maximum((mu + d1)[:, None, :] - d2[None, :, :], 0.0))
    uu = jnp.sum(jnp.maximum((ms + d1n)[:, None, :] - d2n[None, :, :], 0.0))

    del mv_s
    vv = jnp.float32(0.0)
    for j in range(_P):
        dist2j = jnp.sum((negT - posT[j][None]) ** 2, axis=1)
        zj = mv[j][None] - dist2j
        hj = jnp.maximum(d1[:, None, :] + zj[None, :, :], 0.0)
        vv = vv + jnp.sum(hj)

    am = jnp.sum(mu) / _B + jnp.sum(mv) / (_B * _P) + jnp.sum(ms) / _B
    part = uv + _W1 * vv + _W3 * uu + _W2 * am

    @pl.when(i == 0)
    def _init():
        out_ref[0, 0] = 0.0

    out_ref[0, 0] += part


_loss_call = pl.pallas_call(
    _loss_body,
    grid=(_NSTEP,),
    in_specs=[
        pl.BlockSpec((_D, _CHUNK), lambda i: (0, i)),
        pl.BlockSpec((_P, _D, _CHUNK), lambda i: (0, 0, i)),
        pl.BlockSpec((_N, _D, _CHUNK), lambda i: (0, 0, i)),
        pl.BlockSpec((_K, _D, _CHUNK), lambda i: (0, 0, i)),
        pl.BlockSpec((_K, _D, _CHUNK), lambda i: (0, 0, i)),
        pl.BlockSpec((_MG, _CHUNK), lambda i: (0, i)),
        pl.BlockSpec((_P, _MG, _CHUNK), lambda i: (0, 0, i)),
        pl.BlockSpec((_MG, _CHUNK), lambda i: (0, i)),
        pl.BlockSpec((1, _CHUNK), lambda i: (0, i)),
        pl.BlockSpec((_P, _CHUNK), lambda i: (0, i)),
    ],
    out_specs=pl.BlockSpec((1, 1), lambda i: (0, 0), memory_space=pltpu.SMEM),
    out_shape=jax.ShapeDtypeStruct((1, 1), jnp.float32),
    scratch_shapes=[pltpu.VMEM((_P, _CHUNK), jnp.float32)],
)


def kernel(train_u, pos_idx, neg_idx, nb_idx, non_nb_idx, u2e, v2e,
           margin_uv, margin_vv, margin_uu):
    u_emb, pos_e, neg_e, nb_e, nnb_e, m_uv, m_vv, m_uu = _make_sc_gather()(
        train_u, pos_idx.reshape(-1), neg_idx.reshape(-1), nb_idx.reshape(-1),
        non_nb_idx.reshape(-1), u2e, v2e,
        margin_uv.reshape(-1, _MG), margin_vv.reshape(-1, _MG),
        margin_uu.reshape(-1, _MG))
    total = _loss_call(
        u_emb.T,
        pos_e.reshape(_B, _P, _D).transpose(1, 2, 0),
        neg_e.reshape(_B, _N, _D).transpose(1, 2, 0),
        nb_e.reshape(_B, _K, _D).transpose(1, 2, 0),
        nnb_e.reshape(_B, _K, _D).transpose(1, 2, 0),
        m_uv.T,
        m_vv.reshape(_B, _P, _MG).transpose(1, 2, 0),
        m_uu.T,
        train_u.reshape(1, _B),
        pos_idx.T,
    )
    return total[0, 0]


# speedup vs baseline: 1.7774x; 1.1945x over previous
import functools

import jax
import jax.numpy as jnp
from jax import lax
from jax.experimental import pallas as pl
from jax.experimental.pallas import tpu as pltpu
from jax.experimental.pallas import tpu_sc as plsc

_B, _P, _N, _K, _D = 1024, 20, 20, 20, 32
_W1, _W2, _W3 = 0.5, 0.1, 0.5
_NC, _NS = 2, 16
_NW = _NC * _NS
_UPW = _B // _NW
_IPW = _B * _P // _NW
_GCH = 128
_NCHK = _IPW // _GCH
_MG = 16
_NV = 100000

_sc_out_type = (
    jax.ShapeDtypeStruct((_B, _D), jnp.float32),
    jax.ShapeDtypeStruct((_B * _P, _D), jnp.float32),
    jax.ShapeDtypeStruct((_B * _N, _D), jnp.float32),
    jax.ShapeDtypeStruct((_B * _K, _D), jnp.float32),
    jax.ShapeDtypeStruct((_B * _K, _D), jnp.float32),
    jax.ShapeDtypeStruct((_B, _MG), jnp.float32),
    jax.ShapeDtypeStruct((_B * _P, _MG), jnp.float32),
    jax.ShapeDtypeStruct((_B, _MG), jnp.float32),
)

_sc_scratch = (
    pltpu.VMEM((_UPW,), jnp.int32),
    pltpu.VMEM((_IPW,), jnp.int32),
    pltpu.VMEM((_IPW,), jnp.int32),
    pltpu.VMEM((_IPW,), jnp.int32),
    pltpu.VMEM((_IPW,), jnp.int32),
    pltpu.VMEM((_UPW,), jnp.int32),
    pltpu.VMEM((_IPW,), jnp.int32),
    pltpu.VMEM((_UPW, _D), jnp.float32),
    pltpu.VMEM((_IPW, _D), jnp.float32),
    pltpu.VMEM((_IPW, _D), jnp.float32),
    pltpu.VMEM((_IPW, _D), jnp.float32),
    pltpu.VMEM((_IPW, _D), jnp.float32),
    pltpu.VMEM((_UPW, _MG), jnp.float32),
    pltpu.VMEM((_IPW, _MG), jnp.float32),
    pltpu.VMEM((_UPW, _MG), jnp.float32),
    pltpu.SemaphoreType.DMA,
)


@functools.cache
def _make_sc_gather():
    mesh = plsc.VectorSubcoreMesh(core_axis_name="c", subcore_axis_name="s")
    return functools.partial(
        pl.kernel, mesh=mesh, out_type=_sc_out_type,
        scratch_types=list(_sc_scratch),
        compiler_params=pltpu.CompilerParams(use_tc_tiling_on_sc=False),
    )(_sc_gather_body)


def _sc_gather_body(tu_h, pos_h, neg_h, nb_h, nnb_h, u2e_h, v2e_h,
                    muv_h, mvv_h, muu_h,
                    u_o, p_o, n_o, nb_o, nnb_o, mu_o, mv_o, ms_o,
                    iu, ip, ineg, inb, innb, iur, ipr,
                    ru, rp, rn, rnb, rnnb, rmu, rmv, rms, sem):
    wid = lax.axis_index("s") * _NC + lax.axis_index("c")
    bu = wid * _UPW
    bi = wid * _IPW

    u2e, v2e = u2e_h, v2e_h
    muv, mvv, muu = muv_h, mvv_h, muu_h

    pltpu.sync_copy(tu_h.at[pl.ds(bu, _UPW)], iu)
    pltpu.sync_copy(pos_h.at[pl.ds(bi, _IPW)], ip)
    pltpu.sync_copy(neg_h.at[pl.ds(bi, _IPW)], ineg)
    pltpu.sync_copy(nb_h.at[pl.ds(bi, _IPW)], inb)
    pltpu.sync_copy(nnb_h.at[pl.ds(bi, _IPW)], innb)

    for t in range(_UPW // 16):
        s = pl.ds(t * 16, 16)
        iur[s] = lax.shift_right_logical(iu[s], 4)
    for t in range(_IPW // 16):
        s = pl.ds(t * 16, 16)
        ipr[s] = lax.shift_right_logical(ip[s], 4)

    copies = [
        pltpu.async_copy(u2e.at[iu], ru, sem),
        pltpu.async_copy(muv.at[iur], rmu, sem),
        pltpu.async_copy(muu.at[iur], rms, sem),
    ]
    for k in range(_NCHK):
        sl = pl.ds(k * _GCH, _GCH)
        copies.append(pltpu.async_copy(v2e.at[ip.at[sl]], rp.at[sl], sem))
        copies.append(pltpu.async_copy(v2e.at[ineg.at[sl]], rn.at[sl], sem))
        copies.append(pltpu.async_copy(u2e.at[inb.at[sl]], rnb.at[sl], sem))
        copies.append(pltpu.async_copy(u2e.at[innb.at[sl]], rnnb.at[sl], sem))
        copies.append(pltpu.async_copy(mvv.at[ipr.at[sl]], rmv.at[sl], sem))
    for c in copies:
        c.wait()

    pltpu.sync_copy(ru, u_o.at[pl.ds(bu, _UPW)])
    pltpu.sync_copy(rp, p_o.at[pl.ds(bi, _IPW)])
    pltpu.sync_copy(rn, n_o.at[pl.ds(bi, _IPW)])
    pltpu.sync_copy(rnb, nb_o.at[pl.ds(bi, _IPW)])
    pltpu.sync_copy(rnnb, nnb_o.at[pl.ds(bi, _IPW)])
    pltpu.sync_copy(rmu, mu_o.at[pl.ds(bu, _UPW)])
    pltpu.sync_copy(rmv, mv_o.at[pl.ds(bi, _IPW)])
    pltpu.sync_copy(rms, ms_o.at[pl.ds(bu, _UPW)])


_CHUNK = 256
_NSTEP = _B // _CHUNK


def _loss_body(u_ref, pos_ref, neg_ref, nb_ref, nnb_ref,
               mu_ref, mv_ref, ms_ref, tu_ref, pi_ref, out_ref, mv_s):
    i = pl.program_id(0)
    def _t3(x, a):
        return x.reshape(_CHUNK, a * _D).T.reshape(a, _D, _CHUNK)

    uT = u_ref[...].T
    posT = _t3(pos_ref[...], _P)
    negT = _t3(neg_ref[...], _N)
    nbT = _t3(nb_ref[...], _K)
    nnbT = _t3(nnb_ref[...], _K)
    tu = tu_ref[...].T
    pi = pi_ref[...].T

    colu = jnp.bitwise_and(tu, _MG - 1)
    iota_u = lax.broadcasted_iota(jnp.int32, (_MG, _CHUNK), 0)
    oh_u = iota_u == colu
    mu = jnp.sum(jnp.where(oh_u, mu_ref[...].T, 0.0), axis=0, keepdims=True)
    ms = jnp.sum(jnp.where(oh_u, ms_ref[...].T, 0.0), axis=0, keepdims=True)
    colp = jnp.bitwise_and(pi, _MG - 1)
    iota_p = lax.broadcasted_iota(jnp.int32, (_P, _MG, _CHUNK), 1)
    oh_p = iota_p == colp[:, None, :]
    mv = jnp.sum(jnp.where(oh_p, jnp.transpose(mv_ref[...], (1, 2, 0)), 0.0), axis=1)

    ub = uT[None]
    d1 = jnp.sum((posT - ub) ** 2, axis=1)
    d2 = jnp.sum((negT - ub) ** 2, axis=1)
    d1n = jnp.sum((nbT - ub) ** 2, axis=1)
    d2n = jnp.sum((nnbT - ub) ** 2, axis=1)

    uv = jnp.sum(jnp.maximum((mu + d1)[:, None, :] - d2[None, :, :], 0.0))
    uu = jnp.sum(jnp.maximum((ms + d1n)[:, None, :] - d2n[None, :, :], 0.0))

    del mv_s
    vv = jnp.float32(0.0)
    for j in range(_P):
        dist2j = jnp.sum((negT - posT[j][None]) ** 2, axis=1)
        zj = mv[j][None] - dist2j
        hj = jnp.maximum(d1[:, None, :] + zj[None, :, :], 0.0)
        vv = vv + jnp.sum(hj)

    am = jnp.sum(mu) / _B + jnp.sum(mv) / (_B * _P) + jnp.sum(ms) / _B
    part = uv + _W1 * vv + _W3 * uu + _W2 * am

    @pl.when(i == 0)
    def _init():
        out_ref[0, 0] = 0.0

    out_ref[0, 0] += part


_loss_call = pl.pallas_call(
    _loss_body,
    grid=(_NSTEP,),
    in_specs=[
        pl.BlockSpec((_CHUNK, _D), lambda i: (i, 0)),
        pl.BlockSpec((_CHUNK, _P, _D), lambda i: (i, 0, 0)),
        pl.BlockSpec((_CHUNK, _N, _D), lambda i: (i, 0, 0)),
        pl.BlockSpec((_CHUNK, _K, _D), lambda i: (i, 0, 0)),
        pl.BlockSpec((_CHUNK, _K, _D), lambda i: (i, 0, 0)),
        pl.BlockSpec((_CHUNK, _MG), lambda i: (i, 0)),
        pl.BlockSpec((_CHUNK, _P, _MG), lambda i: (i, 0, 0)),
        pl.BlockSpec((_CHUNK, _MG), lambda i: (i, 0)),
        pl.BlockSpec((_CHUNK, 1), lambda i: (i, 0)),
        pl.BlockSpec((_CHUNK, _P), lambda i: (i, 0)),
    ],
    out_specs=pl.BlockSpec((1, 1), lambda i: (0, 0), memory_space=pltpu.SMEM),
    out_shape=jax.ShapeDtypeStruct((1, 1), jnp.float32),
    scratch_shapes=[pltpu.VMEM((_P, _CHUNK), jnp.float32)],
)


def kernel(train_u, pos_idx, neg_idx, nb_idx, non_nb_idx, u2e, v2e,
           margin_uv, margin_vv, margin_uu):
    u_emb, pos_e, neg_e, nb_e, nnb_e, m_uv, m_vv, m_uu = _make_sc_gather()(
        train_u, pos_idx.reshape(-1), neg_idx.reshape(-1), nb_idx.reshape(-1),
        non_nb_idx.reshape(-1), u2e, v2e,
        margin_uv.reshape(-1, _MG), margin_vv.reshape(-1, _MG),
        margin_uu.reshape(-1, _MG))
    total = _loss_call(
        u_emb,
        pos_e.reshape(_B, _P, _D),
        neg_e.reshape(_B, _N, _D),
        nb_e.reshape(_B, _K, _D),
        nnb_e.reshape(_B, _K, _D),
        m_uv,
        m_vv.reshape(_B, _P, _MG),
        m_uu,
        train_u.reshape(_B, 1),
        pos_idx,
    )
    return total[0, 0]


# speedup vs baseline: 2.2522x; 1.2671x over previous
import functools

import jax
import jax.numpy as jnp
from jax import lax
from jax.experimental import pallas as pl
from jax.experimental.pallas import tpu as pltpu
from jax.experimental.pallas import tpu_sc as plsc

_B, _P, _N, _K, _D = 1024, 20, 20, 20, 32
_W1, _W2, _W3 = 0.5, 0.1, 0.5
_NC, _NS = 2, 16
_NW = _NC * _NS
_UPW = _B // _NW
_IPW = _B * _P // _NW
_GCH = 128
_NCHK = _IPW // _GCH
_MG = 16
_NV = 100000

_sc_out_type = (
    jax.ShapeDtypeStruct((_B, _D), jnp.float32),
    jax.ShapeDtypeStruct((_B * _P, _D), jnp.float32),
    jax.ShapeDtypeStruct((_B * _N, _D), jnp.float32),
    jax.ShapeDtypeStruct((_B * _K, _D), jnp.float32),
    jax.ShapeDtypeStruct((_B * _K, _D), jnp.float32),
    jax.ShapeDtypeStruct((_B, _MG), jnp.float32),
    jax.ShapeDtypeStruct((_B * _P, _MG), jnp.float32),
    jax.ShapeDtypeStruct((_B, _MG), jnp.float32),
)

_sc_scratch = (
    pltpu.VMEM((_UPW,), jnp.int32),
    pltpu.VMEM((_IPW,), jnp.int32),
    pltpu.VMEM((_IPW,), jnp.int32),
    pltpu.VMEM((_IPW,), jnp.int32),
    pltpu.VMEM((_IPW,), jnp.int32),
    pltpu.VMEM((_UPW,), jnp.int32),
    pltpu.VMEM((_IPW,), jnp.int32),
    pltpu.VMEM((_UPW, _D), jnp.float32),
    pltpu.VMEM((_IPW, _D), jnp.float32),
    pltpu.VMEM((_IPW, _D), jnp.float32),
    pltpu.VMEM((_IPW, _D), jnp.float32),
    pltpu.VMEM((_IPW, _D), jnp.float32),
    pltpu.VMEM((_UPW, _MG), jnp.float32),
    pltpu.VMEM((_IPW, _MG), jnp.float32),
    pltpu.VMEM((_UPW, _MG), jnp.float32),
    pltpu.SemaphoreType.DMA,
)


@functools.cache
def _make_sc_gather():
    mesh = plsc.VectorSubcoreMesh(core_axis_name="c", subcore_axis_name="s")
    return functools.partial(
        pl.kernel, mesh=mesh, out_type=_sc_out_type,
        scratch_types=list(_sc_scratch),
        compiler_params=pltpu.CompilerParams(use_tc_tiling_on_sc=False),
    )(_sc_gather_body)


def _sc_gather_body(tu_h, pos_h, neg_h, nb_h, nnb_h, u2e_h, v2e_h,
                    muv_h, mvv_h, muu_h,
                    u_o, p_o, n_o, nb_o, nnb_o, mu_o, mv_o, ms_o,
                    iu, ip, ineg, inb, innb, iur, ipr,
                    ru, rp, rn, rnb, rnnb, rmu, rmv, rms, sem):
    wid = lax.axis_index("s") * _NC + lax.axis_index("c")
    bu = wid * _UPW
    bi = wid * _IPW

    u2e, v2e = u2e_h, v2e_h
    muv, mvv, muu = muv_h, mvv_h, muu_h

    pltpu.sync_copy(tu_h.at[pl.ds(bu, _UPW)], iu)
    pltpu.sync_copy(pos_h.at[pl.ds(bi, _IPW)], ip)
    pltpu.sync_copy(neg_h.at[pl.ds(bi, _IPW)], ineg)
    pltpu.sync_copy(nb_h.at[pl.ds(bi, _IPW)], inb)
    pltpu.sync_copy(nnb_h.at[pl.ds(bi, _IPW)], innb)

    for t in range(_UPW // 16):
        s = pl.ds(t * 16, 16)
        iur[s] = lax.shift_right_logical(iu[s], 4)
    for t in range(_IPW // 16):
        s = pl.ds(t * 16, 16)
        ipr[s] = lax.shift_right_logical(ip[s], 4)

    copies = [
        pltpu.async_copy(u2e.at[iu], ru, sem),
        pltpu.async_copy(muv.at[iur], rmu, sem),
        pltpu.async_copy(muu.at[iur], rms, sem),
    ]
    for k in range(_NCHK):
        sl = pl.ds(k * _GCH, _GCH)
        copies.append(pltpu.async_copy(v2e.at[ip.at[sl]], rp.at[sl], sem))
        copies.append(pltpu.async_copy(v2e.at[ineg.at[sl]], rn.at[sl], sem))
        copies.append(pltpu.async_copy(u2e.at[inb.at[sl]], rnb.at[sl], sem))
        copies.append(pltpu.async_copy(u2e.at[innb.at[sl]], rnnb.at[sl], sem))
        copies.append(pltpu.async_copy(mvv.at[ipr.at[sl]], rmv.at[sl], sem))
    for c in copies:
        c.wait()

    pltpu.sync_copy(ru, u_o.at[pl.ds(bu, _UPW)])
    pltpu.sync_copy(rp, p_o.at[pl.ds(bi, _IPW)])
    pltpu.sync_copy(rn, n_o.at[pl.ds(bi, _IPW)])
    pltpu.sync_copy(rnb, nb_o.at[pl.ds(bi, _IPW)])
    pltpu.sync_copy(rnnb, nnb_o.at[pl.ds(bi, _IPW)])
    pltpu.sync_copy(rmu, mu_o.at[pl.ds(bu, _UPW)])
    pltpu.sync_copy(rmv, mv_o.at[pl.ds(bi, _IPW)])
    pltpu.sync_copy(rms, ms_o.at[pl.ds(bu, _UPW)])


_CHUNK = 256
_NSTEP = _B // _CHUNK


def _loss_body(u_ref, pos_ref, neg_ref, nb_ref, nnb_ref,
               mu_ref, mv_ref, ms_ref, tu_ref, pi_ref, out_ref, mv_s):
    i = pl.program_id(0)
    def _t3(x, a):
        return x.T.reshape(a, _D, _CHUNK)

    uT = u_ref[...].T
    posT = _t3(pos_ref[...], _P)
    negT = _t3(neg_ref[...], _N)
    nbT = _t3(nb_ref[...], _K)
    nnbT = _t3(nnb_ref[...], _K)
    tu = tu_ref[...].T
    pi = pi_ref[...].T

    colu = jnp.bitwise_and(tu, _MG - 1)
    iota_u = lax.broadcasted_iota(jnp.int32, (_MG, _CHUNK), 0)
    oh_u = iota_u == colu
    mu = jnp.sum(jnp.where(oh_u, mu_ref[...].T, 0.0), axis=0, keepdims=True)
    ms = jnp.sum(jnp.where(oh_u, ms_ref[...].T, 0.0), axis=0, keepdims=True)
    colp = jnp.bitwise_and(pi, _MG - 1)
    iota_p = lax.broadcasted_iota(jnp.int32, (_P, _MG, _CHUNK), 1)
    oh_p = iota_p == colp[:, None, :]
    mvr = mv_ref[...].T.reshape(_P, _MG, _CHUNK)
    mv = jnp.sum(jnp.where(oh_p, mvr, 0.0), axis=1)

    ub = uT[None]
    d1 = jnp.sum((posT - ub) ** 2, axis=1)
    d2 = jnp.sum((negT - ub) ** 2, axis=1)
    d1n = jnp.sum((nbT - ub) ** 2, axis=1)
    d2n = jnp.sum((nnbT - ub) ** 2, axis=1)

    uv = jnp.sum(jnp.maximum((mu + d1)[:, None, :] - d2[None, :, :], 0.0))
    uu = jnp.sum(jnp.maximum((ms + d1n)[:, None, :] - d2n[None, :, :], 0.0))

    del mv_s
    vv = jnp.float32(0.0)
    for j in range(_P):
        dist2j = jnp.sum((negT - posT[j][None]) ** 2, axis=1)
        zj = mv[j][None] - dist2j
        hj = jnp.maximum(d1[:, None, :] + zj[None, :, :], 0.0)
        vv = vv + jnp.sum(hj)

    am = jnp.sum(mu) / _B + jnp.sum(mv) / (_B * _P) + jnp.sum(ms) / _B
    part = uv + _W1 * vv + _W3 * uu + _W2 * am

    @pl.when(i == 0)
    def _init():
        out_ref[0, 0] = 0.0

    out_ref[0, 0] += part


_loss_call = pl.pallas_call(
    _loss_body,
    grid=(_NSTEP,),
    in_specs=[
        pl.BlockSpec((_CHUNK, _D), lambda i: (i, 0)),
        pl.BlockSpec((_CHUNK, _P * _D), lambda i: (i, 0)),
        pl.BlockSpec((_CHUNK, _N * _D), lambda i: (i, 0)),
        pl.BlockSpec((_CHUNK, _K * _D), lambda i: (i, 0)),
        pl.BlockSpec((_CHUNK, _K * _D), lambda i: (i, 0)),
        pl.BlockSpec((_CHUNK, _MG), lambda i: (i, 0)),
        pl.BlockSpec((_CHUNK, _P * _MG), lambda i: (i, 0)),
        pl.BlockSpec((_CHUNK, _MG), lambda i: (i, 0)),
        pl.BlockSpec((_CHUNK, 1), lambda i: (i, 0)),
        pl.BlockSpec((_CHUNK, _P), lambda i: (i, 0)),
    ],
    out_specs=pl.BlockSpec((1, 1), lambda i: (0, 0), memory_space=pltpu.SMEM),
    out_shape=jax.ShapeDtypeStruct((1, 1), jnp.float32),
    scratch_shapes=[pltpu.VMEM((_P, _CHUNK), jnp.float32)],
)


def kernel(train_u, pos_idx, neg_idx, nb_idx, non_nb_idx, u2e, v2e,
           margin_uv, margin_vv, margin_uu):
    u_emb, pos_e, neg_e, nb_e, nnb_e, m_uv, m_vv, m_uu = _make_sc_gather()(
        train_u, pos_idx.reshape(-1), neg_idx.reshape(-1), nb_idx.reshape(-1),
        non_nb_idx.reshape(-1), u2e, v2e,
        margin_uv.reshape(-1, _MG), margin_vv.reshape(-1, _MG),
        margin_uu.reshape(-1, _MG))
    total = _loss_call(
        u_emb,
        pos_e.reshape(_B, _P * _D),
        neg_e.reshape(_B, _N * _D),
        nb_e.reshape(_B, _K * _D),
        nnb_e.reshape(_B, _K * _D),
        m_uv,
        m_vv.reshape(_B, _P * _MG),
        m_uu,
        train_u.reshape(_B, 1),
        pos_idx,
    )
    return total[0, 0]
